# baseline (speedup 1.0000x reference)
import functools

import jax
import jax.numpy as jnp
from jax import lax
from jax.experimental import pallas as pl
from jax.experimental.pallas import tpu as pltpu
from jax.experimental.pallas import tpu_sc as plsc

IN_SIZE = 128
OUT_SIZE = 64
BATCH = 16384
NUM_ROWS = 1000000

_info = plsc.get_sparse_core_info()
_NC, _NS = _info.num_cores, _info.num_subcores
_NW = _NC * _NS
_BPW = BATCH // _NW
_CH = 128
_NCH = _BPW // _CH

_RB = 2048
_NBLK = (NUM_ROWS + _RB - 1) // _RB
_PACKED_ROWS = _NBLK * (_RB // 2)
_HB = _RB // 2


def _relayout(t_ref, eye_ref, o_ref):
    lo = lax.dot_general(
        t_ref[:, 0:_HB], eye_ref[...],
        (((0,), (0,)), ((), ())),
        preferred_element_type=jnp.float32,
    )
    hi = lax.dot_general(
        t_ref[:, _HB:_RB], eye_ref[...],
        (((0,), (0,)), ((), ())),
        preferred_element_type=jnp.float32,
    )
    o_ref[...] = jnp.concatenate([lo, hi], axis=1)


def _pack_table(embT, eye):
    return pl.pallas_call(
        _relayout,
        grid=(_NBLK,),
        in_specs=[
            pl.BlockSpec((OUT_SIZE, _RB), lambda i: (0, i)),
            pl.BlockSpec((OUT_SIZE, OUT_SIZE), lambda i: (0, 0)),
        ],
        out_specs=pl.BlockSpec((_HB, 128), lambda i: (i, 0)),
        out_shape=jax.ShapeDtypeStruct((_PACKED_ROWS, 128), jnp.float32),
    )(embT, eye)


@functools.partial(
    pl.kernel,
    mesh=plsc.VectorSubcoreMesh(core_axis_name="c", subcore_axis_name="s"),
    out_type=jax.ShapeDtypeStruct((BATCH, 128), jnp.float32),
    scratch_types=[
        pltpu.VMEM((_BPW,), jnp.int32),
        pltpu.VMEM((_CH, 128), jnp.float32),
        pltpu.SemaphoreType.DMA,
    ],
    compiler_params=pltpu.CompilerParams(needs_layout_passes=False),
)
def _sc_gather(table_hbm, blk_hbm, out_hbm, blk_v, rows_v, sem):
    wid = lax.axis_index("s") * _NC + lax.axis_index("c")
    base = wid * _BPW
    pltpu.sync_copy(blk_hbm.at[pl.ds(base, _BPW)], blk_v)
    for ch in range(_NCH):
        pltpu.async_copy(
            table_hbm.at[blk_v.at[pl.ds(ch * _CH, _CH)]], rows_v, sem
        ).wait()
        pltpu.sync_copy(rows_v, out_hbm.at[pl.ds(base + ch * _CH, _CH)])


def _fc_mul(x_ref, w_ref, g_ref, p_ref, o_ref):
    fc = lax.dot_general(
        x_ref[...], w_ref[...],
        (((1,), (1,)), ((), ())),
        preferred_element_type=jnp.float32,
    )
    sel = jnp.where(
        p_ref[...] != 0,
        g_ref[:, OUT_SIZE:2 * OUT_SIZE],
        g_ref[:, 0:OUT_SIZE],
    )
    o_ref[...] = sel * fc


_BLK = 2048


def kernel(x, id, fc_weight, emb_weight):
    id32 = id.astype(jnp.int32)
    blk = id32 >> 1
    par = (id32 & 1).reshape(BATCH, 1)
    packed = emb_weight.reshape(NUM_ROWS // 2, 128)
    g = _sc_gather(packed, blk)
    out = pl.pallas_call(
        _fc_mul,
        grid=(BATCH // _BLK,),
        in_specs=[
            pl.BlockSpec((_BLK, IN_SIZE), lambda i: (i, 0)),
            pl.BlockSpec((OUT_SIZE, IN_SIZE), lambda i: (0, 0)),
            pl.BlockSpec((_BLK, 128), lambda i: (i, 0)),
            pl.BlockSpec((_BLK, 1), lambda i: (i, 0)),
        ],
        out_specs=pl.BlockSpec((_BLK, OUT_SIZE), lambda i: (i, 0)),
        out_shape=jax.ShapeDtypeStruct((BATCH, OUT_SIZE), jnp.float32),
    )(x, fc_weight, g, par)
    return out


# speedup vs baseline: 1.7151x; 1.7151x over previous
import functools

import jax
import jax.numpy as jnp
from jax import lax
from jax.experimental import pallas as pl
from jax.experimental.pallas import tpu as pltpu
from jax.experimental.pallas import tpu_sc as plsc

IN_SIZE = 128
OUT_SIZE = 64
BATCH = 16384
NUM_ROWS = 1000000

_info = plsc.get_sparse_core_info()
_NC, _NS = _info.num_cores, _info.num_subcores
_NW = _NC * _NS
_BPW = BATCH // _NW
_CH = 128
_NCH = _BPW // _CH

_RB = 4096
_NBLK = (NUM_ROWS + _RB - 1) // _RB
_PACKED_ROWS = _NBLK * (_RB // 2)
_HB = _RB // 2


def _relayout(t_ref, eye_ref, o_ref):
    lo = lax.dot_general(
        t_ref[:, 0:_HB], eye_ref[...],
        (((0,), (0,)), ((), ())),
        preferred_element_type=jnp.float32,
    )
    hi = lax.dot_general(
        t_ref[:, _HB:_RB], eye_ref[...],
        (((0,), (0,)), ((), ())),
        preferred_element_type=jnp.float32,
    )
    o_ref[...] = jnp.concatenate([lo, hi], axis=1)


def _pack_table(embT, eye):
    return pl.pallas_call(
        _relayout,
        grid=(_NBLK,),
        in_specs=[
            pl.BlockSpec((OUT_SIZE, _RB), lambda i: (0, i)),
            pl.BlockSpec((OUT_SIZE, OUT_SIZE), lambda i: (0, 0)),
        ],
        out_specs=pl.BlockSpec((_HB, 128), lambda i: (i, 0)),
        out_shape=jax.ShapeDtypeStruct((_PACKED_ROWS, 128), jnp.float32),
        compiler_params=pltpu.CompilerParams(
            dimension_semantics=("parallel",),
        ),
    )(embT, eye)


@functools.partial(
    pl.kernel,
    mesh=plsc.VectorSubcoreMesh(core_axis_name="c", subcore_axis_name="s"),
    out_type=jax.ShapeDtypeStruct((BATCH, 128), jnp.float32),
    scratch_types=[
        pltpu.VMEM((_BPW,), jnp.int32),
        pltpu.VMEM((_CH, 128), jnp.float32),
        pltpu.SemaphoreType.DMA,
    ],
    compiler_params=pltpu.CompilerParams(needs_layout_passes=False),
)
def _sc_gather(table_hbm, blk_hbm, out_hbm, blk_v, rows_v, sem):
    wid = lax.axis_index("s") * _NC + lax.axis_index("c")
    base = wid * _BPW
    pltpu.sync_copy(blk_hbm.at[pl.ds(base, _BPW)], blk_v)
    for ch in range(_NCH):
        pltpu.async_copy(
            table_hbm.at[blk_v.at[pl.ds(ch * _CH, _CH)]], rows_v, sem
        ).wait()
        pltpu.sync_copy(rows_v, out_hbm.at[pl.ds(base + ch * _CH, _CH)])


def _fc_mul(x_ref, w_ref, g_ref, p_ref, o_ref):
    fc = lax.dot_general(
        x_ref[...], w_ref[...],
        (((1,), (1,)), ((), ())),
        preferred_element_type=jnp.float32,
    )
    sel = jnp.where(
        p_ref[...] != 0,
        g_ref[:, OUT_SIZE:2 * OUT_SIZE],
        g_ref[:, 0:OUT_SIZE],
    )
    o_ref[...] = sel * fc


_BLK = 2048


def kernel(x, id, fc_weight, emb_weight):
    id32 = id.astype(jnp.int32)
    blk = ((id32 >> 12) << 11) | (id32 & (_HB - 1))
    par = ((id32 >> 11) & 1).reshape(BATCH, 1)
    packed = _pack_table(emb_weight.T, jnp.eye(OUT_SIZE, dtype=jnp.float32))
    g = _sc_gather(packed, blk)
    out = pl.pallas_call(
        _fc_mul,
        grid=(BATCH // _BLK,),
        in_specs=[
            pl.BlockSpec((_BLK, IN_SIZE), lambda i: (i, 0)),
            pl.BlockSpec((OUT_SIZE, IN_SIZE), lambda i: (0, 0)),
            pl.BlockSpec((_BLK, 128), lambda i: (i, 0)),
            pl.BlockSpec((_BLK, 1), lambda i: (i, 0)),
        ],
        out_specs=pl.BlockSpec((_BLK, OUT_SIZE), lambda i: (i, 0)),
        out_shape=jax.ShapeDtypeStruct((BATCH, OUT_SIZE), jnp.float32),
    )(x, fc_weight, g, par)
    return out


# speedup vs baseline: 1.7188x; 1.0022x over previous
import functools

import jax
import jax.numpy as jnp
from jax import lax
from jax.experimental import pallas as pl
from jax.experimental.pallas import tpu as pltpu
from jax.experimental.pallas import tpu_sc as plsc

IN_SIZE = 128
OUT_SIZE = 64
BATCH = 16384
NUM_ROWS = 1000000

_info = plsc.get_sparse_core_info()
_NC, _NS = _info.num_cores, _info.num_subcores
_NW = _NC * _NS
_BPW = BATCH // _NW
_CH = 128
_NCH = _BPW // _CH

_RB = 4096
_NBLK = (NUM_ROWS + _RB - 1) // _RB
_PACKED_ROWS = _NBLK * (_RB // 2)
_HB = _RB // 2


def _relayout(t_ref, eye_ref, o_ref):
    lo = jnp.transpose(t_ref[:, 0:_HB], (1, 0))
    hi = jnp.transpose(t_ref[:, _HB:_RB], (1, 0))
    o_ref[...] = jnp.concatenate([lo, hi], axis=1)


def _pack_table(embT, eye):
    return pl.pallas_call(
        _relayout,
        grid=(_NBLK,),
        in_specs=[
            pl.BlockSpec((OUT_SIZE, _RB), lambda i: (0, i)),
            pl.BlockSpec((OUT_SIZE, OUT_SIZE), lambda i: (0, 0)),
        ],
        out_specs=pl.BlockSpec((_HB, 128), lambda i: (i, 0)),
        out_shape=jax.ShapeDtypeStruct((_PACKED_ROWS, 128), jnp.float32),
        compiler_params=pltpu.CompilerParams(
            dimension_semantics=("parallel",),
        ),
    )(embT, eye)


@functools.partial(
    pl.kernel,
    mesh=plsc.VectorSubcoreMesh(core_axis_name="c", subcore_axis_name="s"),
    out_type=jax.ShapeDtypeStruct((BATCH, 128), jnp.float32),
    scratch_types=[
        pltpu.VMEM((_BPW,), jnp.int32),
        pltpu.VMEM((_CH, 128), jnp.float32),
        pltpu.SemaphoreType.DMA,
    ],
    compiler_params=pltpu.CompilerParams(needs_layout_passes=False),
)
def _sc_gather(table_hbm, blk_hbm, out_hbm, blk_v, rows_v, sem):
    wid = lax.axis_index("s") * _NC + lax.axis_index("c")
    base = wid * _BPW
    pltpu.sync_copy(blk_hbm.at[pl.ds(base, _BPW)], blk_v)
    for ch in range(_NCH):
        pltpu.async_copy(
            table_hbm.at[blk_v.at[pl.ds(ch * _CH, _CH)]], rows_v, sem
        ).wait()
        pltpu.sync_copy(rows_v, out_hbm.at[pl.ds(base + ch * _CH, _CH)])


def _fc_mul(x_ref, w_ref, g_ref, p_ref, o_ref):
    fc = lax.dot_general(
        x_ref[...], w_ref[...],
        (((1,), (1,)), ((), ())),
        preferred_element_type=jnp.float32,
    )
    sel = jnp.where(
        p_ref[...] != 0,
        g_ref[:, OUT_SIZE:2 * OUT_SIZE],
        g_ref[:, 0:OUT_SIZE],
    )
    o_ref[...] = sel * fc


_BLK = 2048


def kernel(x, id, fc_weight, emb_weight):
    id32 = id.astype(jnp.int32)
    blk = ((id32 >> 12) << 11) | (id32 & (_HB - 1))
    par = ((id32 >> 11) & 1).reshape(BATCH, 1)
    packed = _pack_table(emb_weight.T, jnp.eye(OUT_SIZE, dtype=jnp.float32))
    g = _sc_gather(packed, blk)
    out = pl.pallas_call(
        _fc_mul,
        grid=(BATCH // _BLK,),
        in_specs=[
            pl.BlockSpec((_BLK, IN_SIZE), lambda i: (i, 0)),
            pl.BlockSpec((OUT_SIZE, IN_SIZE), lambda i: (0, 0)),
            pl.BlockSpec((_BLK, 128), lambda i: (i, 0)),
            pl.BlockSpec((_BLK, 1), lambda i: (i, 0)),
        ],
        out_specs=pl.BlockSpec((_BLK, OUT_SIZE), lambda i: (i, 0)),
        out_shape=jax.ShapeDtypeStruct((BATCH, OUT_SIZE), jnp.float32),
    )(x, fc_weight, g, par)
    return out


# speedup vs baseline: 1.7202x; 1.0008x over previous
import functools

import jax
import jax.numpy as jnp
from jax import lax
from jax.experimental import pallas as pl
from jax.experimental.pallas import tpu as pltpu
from jax.experimental.pallas import tpu_sc as plsc

IN_SIZE = 128
OUT_SIZE = 64
BATCH = 16384
NUM_ROWS = 1000000

_info = plsc.get_sparse_core_info()
_NC, _NS = _info.num_cores, _info.num_subcores
_NW = _NC * _NS
_BPW = BATCH // _NW
_CH = 128
_NCH = _BPW // _CH

_RB = 4096
_NBLK = (NUM_ROWS + _RB - 1) // _RB
_PACKED_ROWS = _NBLK * (_RB // 2)
_HB = _RB // 2


def _relayout(t_ref, eye_ref, o_ref):
    lo = jnp.transpose(t_ref[:, 0:_HB], (1, 0))
    hi = jnp.transpose(t_ref[:, _HB:_RB], (1, 0))
    o_ref[...] = jnp.concatenate([lo, hi], axis=1)


def _pack_table(embT, eye):
    return pl.pallas_call(
        _relayout,
        grid=(5, _NBLK // 5),
        in_specs=[
            pl.BlockSpec((OUT_SIZE, _RB), lambda i, j: (0, i * (_NBLK // 5) + j)),
            pl.BlockSpec((OUT_SIZE, OUT_SIZE), lambda i, j: (0, 0)),
        ],
        out_specs=pl.BlockSpec(
            (_HB, 128), lambda i, j: (i * (_NBLK // 5) + j, 0)
        ),
        out_shape=jax.ShapeDtypeStruct((_PACKED_ROWS, 128), jnp.float32),
        compiler_params=pltpu.CompilerParams(
            dimension_semantics=("parallel", "arbitrary"),
        ),
    )(embT, eye)


@functools.partial(
    pl.kernel,
    mesh=plsc.VectorSubcoreMesh(core_axis_name="c", subcore_axis_name="s"),
    out_type=jax.ShapeDtypeStruct((BATCH, 128), jnp.float32),
    scratch_types=[
        pltpu.VMEM((_BPW,), jnp.int32),
        pltpu.VMEM((_CH, 128), jnp.float32),
        pltpu.SemaphoreType.DMA,
    ],
    compiler_params=pltpu.CompilerParams(needs_layout_passes=False),
)
def _sc_gather(table_hbm, blk_hbm, out_hbm, blk_v, rows_v, sem):
    wid = lax.axis_index("s") * _NC + lax.axis_index("c")
    base = wid * _BPW
    pltpu.sync_copy(blk_hbm.at[pl.ds(base, _BPW)], blk_v)
    for ch in range(_NCH):
        pltpu.async_copy(
            table_hbm.at[blk_v.at[pl.ds(ch * _CH, _CH)]], rows_v, sem
        ).wait()
        pltpu.sync_copy(rows_v, out_hbm.at[pl.ds(base + ch * _CH, _CH)])


def _fc_mul(x_ref, w_ref, g_ref, p_ref, o_ref):
    fc = lax.dot_general(
        x_ref[...], w_ref[...],
        (((1,), (1,)), ((), ())),
        preferred_element_type=jnp.float32,
    )
    sel = jnp.where(
        p_ref[...] != 0,
        g_ref[:, OUT_SIZE:2 * OUT_SIZE],
        g_ref[:, 0:OUT_SIZE],
    )
    o_ref[...] = sel * fc


_BLK = 2048


def kernel(x, id, fc_weight, emb_weight):
    id32 = id.astype(jnp.int32)
    blk = ((id32 >> 12) << 11) | (id32 & (_HB - 1))
    par = ((id32 >> 11) & 1).reshape(BATCH, 1)
    packed = _pack_table(emb_weight.T, jnp.eye(OUT_SIZE, dtype=jnp.float32))
    g = _sc_gather(packed, blk)
    out = pl.pallas_call(
        _fc_mul,
        grid=(BATCH // _BLK,),
        in_specs=[
            pl.BlockSpec((_BLK, IN_SIZE), lambda i: (i, 0)),
            pl.BlockSpec((OUT_SIZE, IN_SIZE), lambda i: (0, 0)),
            pl.BlockSpec((_BLK, 128), lambda i: (i, 0)),
            pl.BlockSpec((_BLK, 1), lambda i: (i, 0)),
        ],
        out_specs=pl.BlockSpec((_BLK, OUT_SIZE), lambda i: (i, 0)),
        out_shape=jax.ShapeDtypeStruct((BATCH, OUT_SIZE), jnp.float32),
    )(x, fc_weight, g, par)
    return out
